# baseline (speedup 1.0000x reference)
import functools

import jax
import jax.numpy as jnp
import numpy as np
from jax import lax
from jax.experimental import pallas as pl
from jax.experimental.pallas import tpu as pltpu
from jax.experimental.pallas import tpu_sc as plsc

_B, _N, _D, _H = 2, 10000, 128, 16
_GDIM = 64
_W2 = 2 * _H
_WP = 128
_NPAD = 10240

_NC, _NS = 2, 16
_NW = _NC * _NS
_C = 128
_ZR = _NPAD // _NS

_mesh = plsc.VectorSubcoreMesh(core_axis_name="c", subcore_axis_name="s")


def _seg_body(table, srcp, dstp, zer, out, idx_s, idx_d, rows0, rows1,
              acc, gsem0, gsem1, ssem0, ssem1):
    c = lax.axis_index("c")
    s = lax.axis_index("s")
    w = s * _NC + c
    hh = idx_s.shape[0]
    pltpu.sync_copy(zer.at[pl.ds(s * _ZR, _ZR), :], acc.at[pl.ds(s * _ZR, _ZR), :])
    plsc.subcore_barrier()

    rows = [rows0, rows1]
    sems = [gsem0, gsem1]
    ssems = [ssem0, ssem1]
    sdesc = [None, None]
    for half in range(2):
        pltpu.sync_copy(srcp.at[w, pl.ds(half * hh, hh)], idx_s)
        pltpu.sync_copy(dstp.at[w, pl.ds(half * hh, hh)], idx_d)
        descs = [None, None]
        if sdesc[0] is not None:
            sdesc[0].wait()
            sdesc[0] = None
        descs[0] = pltpu.async_copy(table.at[idx_s.at[0]], rows[0], sems[0])
        for j in range(hh):
            b = j % 2
            nb = (j + 1) % 2
            if j + 1 < hh:
                if sdesc[nb] is not None:
                    sdesc[nb].wait()
                descs[nb] = pltpu.async_copy(table.at[idx_s.at[j + 1]], rows[nb], sems[nb])
            descs[b].wait()
            sdesc[b] = pltpu.async_copy(rows[b], acc.at[idx_d.at[j]], ssems[b], add=True)
        for b in range(2):
            if sdesc[b] is not None:
                sdesc[b].wait()
                sdesc[b] = None

    plsc.subcore_barrier()
    pltpu.sync_copy(acc.at[pl.ds(s * _ZR, _ZR), :],
                    out.at[pl.ds(c * _NPAD + s * _ZR, _ZR), :])


def _deg_body(dstp, zer1, out, idx_d, ones_v, acc1):
    c = lax.axis_index("c")
    s = lax.axis_index("s")
    w = s * _NC + c
    cpw = idx_d.shape[0]
    pltpu.sync_copy(zer1.at[pl.ds(s * _ZR, _ZR)], acc1.at[pl.ds(s * _ZR, _ZR)])
    pltpu.sync_copy(dstp.at[w], idx_d)
    for i in range(_C // 16):
        ones_v[pl.ds(i * 16, 16)] = jnp.ones((16,), jnp.float32)
    plsc.subcore_barrier()

    def step(j, carry):
        pltpu.sync_copy(ones_v, acc1.at[idx_d.at[j]], add=True)
        return carry

    lax.fori_loop(0, cpw, step, 0)
    plsc.subcore_barrier()

    @pl.when(s == 0)
    def _():
        pltpu.sync_copy(acc1, out.at[pl.ds(c * _NPAD, _NPAD)])


def _make_seg(cpw):
    return pl.kernel(
        _seg_body,
        out_type=jax.ShapeDtypeStruct((_NC * _NPAD, _WP), jnp.float32),
        mesh=_mesh,
        scratch_types=[
            pltpu.VMEM((cpw // 2, _C), jnp.int32),
            pltpu.VMEM((cpw // 2, _C), jnp.int32),
            pltpu.VMEM((_C, _WP), jnp.float32),
            pltpu.VMEM((_C, _WP), jnp.float32),
            pltpu.VMEM_SHARED((_NPAD, _WP), jnp.float32),
            pltpu.SemaphoreType.DMA,
            pltpu.SemaphoreType.DMA,
            pltpu.SemaphoreType.DMA,
            pltpu.SemaphoreType.DMA,
        ],
    )


def _make_deg(cpw):
    return pl.kernel(
        _deg_body,
        out_type=jax.ShapeDtypeStruct((_NC * _NPAD,), jnp.float32),
        mesh=_mesh,
        scratch_types=[
            pltpu.VMEM((cpw, _C), jnp.int32),
            pltpu.VMEM((_C,), jnp.float32),
            pltpu.VMEM_SHARED((_NPAD,), jnp.float32),
        ],
    )


_NB, _R = 10, _N // 10


def _k0_body(x_ref, w_ref, degp_ref, tab_ref, dinv_ref):
    deg = degp_ref[0] + degp_ref[1] + 1.0
    dinv = lax.rsqrt(deg)
    w = w_ref[...]
    h0 = jnp.dot(x_ref[0], w, preferred_element_type=jnp.float32, precision=lax.Precision.HIGHEST)
    h1 = jnp.dot(x_ref[1], w, preferred_element_type=jnp.float32, precision=lax.Precision.HIGHEST)
    t32 = jnp.concatenate([h0 * dinv, h1 * dinv], axis=1)
    tab_ref[...] = jnp.concatenate([t32, t32, t32, t32], axis=1)
    dinv_ref[...] = dinv


def _kmid_body(sp_ref, tp_ref, dinv_ref, b_ref, wbd_ref, out_ref):
    dinv = dinv_ref[...]
    s32 = (sp_ref[0] + sp_ref[1] + tp_ref[...])[:, :_W2]
    feat = dinv * s32 + b_ref[...]
    h = jnp.dot(feat, wbd_ref[...], preferred_element_type=jnp.float32, precision=lax.Precision.HIGHEST) * dinv
    out_ref[...] = jnp.concatenate([h, h, h, h], axis=1)


def _k4_body(sp_ref, tp_ref, dinv_ref, b_ref, wg1_ref, bg1_ref, wg2_ref, bg2_ref,
             wsrc_ref, wact_ref, wdst_ref, wg_ref, be1_ref, we2_ref, be2_ref,
             we3_ref, be3_ref, agent_ref, v_ref, feat_ref):
    s32 = (sp_ref[0, :_N, :] + sp_ref[1, :_N, :] + tp_ref[...])[:, :_W2]
    feat = dinv_ref[...] * s32 + b_ref[...]
    feat_ref[...] = feat
    m = jnp.sum(feat, axis=0, keepdims=True) * (1.0 / _N)
    g = jnp.concatenate([m[:, :_H], m[:, _H:]], axis=0)
    g1 = jnp.tanh(jnp.dot(g, wg1_ref[...], preferred_element_type=jnp.float32, precision=lax.Precision.HIGHEST) + bg1_ref[...])
    g2 = jnp.tanh(jnp.dot(g1, wg2_ref[...], preferred_element_type=jnp.float32, precision=lax.Precision.HIGHEST) + bg2_ref[...])
    gterm = jnp.dot(g2, wg_ref[...], preferred_element_type=jnp.float32, precision=lax.Precision.HIGHEST)

    srows, drows = [], []
    for b in range(_B):
        for k in range(8):
            si = agent_ref[b, k, 0]
            di = agent_ref[b, k, 1]
            srows.append(feat_ref[pl.ds(si, 1), pl.ds(b * _H, _H)])
            drows.append(feat_ref[pl.ds(di, 1), pl.ds(b * _H, _H)])
    src_f = jnp.concatenate(srows, axis=0)
    dst_f = jnp.concatenate(drows, axis=0)
    gt = jnp.concatenate(
        [jnp.broadcast_to(gterm[0:1, :], (8, _H)),
         jnp.broadcast_to(gterm[1:2, :], (8, _H))], axis=0)
    p = (jnp.dot(src_f, wsrc_ref[...], preferred_element_type=jnp.float32, precision=lax.Precision.HIGHEST)
         + jnp.dot(dst_f, wdst_ref[...], preferred_element_type=jnp.float32, precision=lax.Precision.HIGHEST)
         + gt + be1_ref[...])
    wact = wact_ref[...]
    vcols = []
    for t in range(3):
        h1 = jnp.tanh(p + wact[t:t + 1, :])
        h2 = jnp.tanh(jnp.dot(h1, we2_ref[...], preferred_element_type=jnp.float32, precision=lax.Precision.HIGHEST) + be2_ref[...])
        vcols.append(jnp.dot(h2, we3_ref[...], preferred_element_type=jnp.float32, precision=lax.Precision.HIGHEST) + be3_ref[...])
    v_ref[...] = jnp.concatenate(vcols, axis=1)


def kernel(x, edge_index, agent_edges, edge_actions, W_gcn, b_gcn, Wg1, bg1,
           Wg2, bg2, We1, be1, We2, be2, We3, be3):
    f32 = jnp.float32
    E = edge_index.shape[1]
    cpw = -(-E // (_NW * _C))
    cpw = -(-cpw // 8) * 8
    EP = _NW * cpw * _C
    padn = EP - E
    dum = _NPAD - _N

    pidx = jnp.arange(padn, dtype=jnp.int32)
    srcp = jnp.concatenate([edge_index[0], pidx % 16]).reshape(_NW, cpw, _C)
    dstp = jnp.concatenate([edge_index[1], _N + (pidx % dum)]).reshape(_NW, cpw, _C)

    zer1 = jnp.zeros((_NPAD,), f32)
    zer = jnp.zeros((_NPAD, _WP), f32)

    deg = _make_deg(cpw)(dstp, zer1).reshape(_NC, _NPAD)
    degp = deg[:, :_N, None]

    k0 = pl.pallas_call(
        _k0_body,
        grid=(_NB,),
        in_specs=[
            pl.BlockSpec((_B, _R, _D), lambda i: (0, i, 0)),
            pl.BlockSpec((_D, _H), lambda i: (0, 0)),
            pl.BlockSpec((_NC, _R, 1), lambda i: (0, i, 0)),
        ],
        out_specs=[
            pl.BlockSpec((_R, _WP), lambda i: (i, 0)),
            pl.BlockSpec((_R, 1), lambda i: (i, 0)),
        ],
        out_shape=[
            jax.ShapeDtypeStruct((_N, _WP), f32),
            jax.ShapeDtypeStruct((_N, 1), f32),
        ],
    )
    table, dinv = k0(x, W_gcn[0], degp)

    seg = _make_seg(cpw)
    kmid = pl.pallas_call(
        _kmid_body,
        grid=(_NB,),
        in_specs=[
            pl.BlockSpec((_NC, _R, _WP), lambda i: (0, i, 0)),
            pl.BlockSpec((_R, _WP), lambda i: (i, 0)),
            pl.BlockSpec((_R, 1), lambda i: (i, 0)),
            pl.BlockSpec((1, _W2), lambda i: (0, 0)),
            pl.BlockSpec((_W2, _W2), lambda i: (0, 0)),
        ],
        out_specs=pl.BlockSpec((_R, _WP), lambda i: (i, 0)),
        out_shape=jax.ShapeDtypeStruct((_N, _WP), f32),
    )

    def run_seg(tab):
        return seg(tab, srcp, dstp, zer).reshape(_NC, _NPAD, _WP)[:, :_N, :]

    for l in range(1, 4):
        sp = run_seg(table)
        wbd = jnp.zeros((_W2, _W2), f32).at[:_H, :_H].set(W_gcn[l]).at[_H:, _H:].set(W_gcn[l])
        b2 = jnp.concatenate([b_gcn[l - 1], b_gcn[l - 1]])[None, :]
        table = kmid(sp, table, dinv, b2, wbd)

    spf = seg(table, srcp, dstp, zer).reshape(_NC, _NPAD, _WP)
    b2 = jnp.concatenate([b_gcn[3], b_gcn[3]])[None, :]

    k4 = pl.pallas_call(
        _k4_body,
        in_specs=[
            pl.BlockSpec((_NC, _NPAD, _WP), lambda: (0, 0, 0)),
            pl.BlockSpec((_N, _WP), lambda: (0, 0)),
            pl.BlockSpec((_N, 1), lambda: (0, 0)),
            pl.BlockSpec((1, _W2), lambda: (0, 0)),
            pl.BlockSpec((_H, 2 * _H), lambda: (0, 0)),
            pl.BlockSpec((1, 2 * _H), lambda: (0, 0)),
            pl.BlockSpec((2 * _H, _GDIM), lambda: (0, 0)),
            pl.BlockSpec((1, _GDIM), lambda: (0, 0)),
            pl.BlockSpec((_H, _H), lambda: (0, 0)),
            pl.BlockSpec((3, _H), lambda: (0, 0)),
            pl.BlockSpec((_H, _H), lambda: (0, 0)),
            pl.BlockSpec((_GDIM, _H), lambda: (0, 0)),
            pl.BlockSpec((1, _H), lambda: (0, 0)),
            pl.BlockSpec((_H, 8), lambda: (0, 0)),
            pl.BlockSpec((1, 8), lambda: (0, 0)),
            pl.BlockSpec((8, 1), lambda: (0, 0)),
            pl.BlockSpec((1, 1), lambda: (0, 0)),
            pl.BlockSpec(memory_space=pltpu.SMEM),
        ],
        out_specs=pl.BlockSpec((2 * 8, 3), lambda: (0, 0)),
        out_shape=jax.ShapeDtypeStruct((2 * 8, 3), f32),
        scratch_shapes=[pltpu.VMEM((_N, _W2), f32)],
    )
    v = k4(spf, table, dinv, b2, Wg1, bg1[None, :], Wg2, bg2[None, :],
           We1[:_H], We1[_H:_H + 3], We1[_H + 3:2 * _H + 3], We1[2 * _H + 3:],
           be1[None, :], We2, be2[None, :], We3, be3[None, :], agent_edges)

    v = v.reshape(_B, 8, 3)
    turn = jnp.arange(3, dtype=jnp.int32)[None, :]
    outs = []
    for bi in range(_B):
        disc = turn * 5 + edge_actions[bi][:, None]
        outs.append(jnp.zeros((15,), f32).at[disc.ravel()].set(v[bi].ravel()))
    return jnp.stack(outs, axis=0)


# speedup vs baseline: 1.0348x; 1.0348x over previous
import functools

import jax
import jax.numpy as jnp
import numpy as np
from jax import lax
from jax.experimental import pallas as pl
from jax.experimental.pallas import tpu as pltpu
from jax.experimental.pallas import tpu_sc as plsc

_B, _N, _D, _H = 2, 10000, 128, 16
_GDIM = 64
_W2 = 2 * _H
_WP = 128
_NPAD = 10240

_NC, _NS = 2, 16
_NW = _NC * _NS
_C = 128
_ZR = _NPAD // _NS

_mesh = plsc.VectorSubcoreMesh(core_axis_name="c", subcore_axis_name="s")


def _seg_body(table, srcp, dstp, zer, out, idx_s, idx_d, rows0, rows1,
              acc, gsem0, gsem1, ssem0, ssem1):
    c = lax.axis_index("c")
    s = lax.axis_index("s")
    w = s * _NC + c
    hh = idx_s.shape[0]
    pltpu.sync_copy(zer.at[pl.ds(s * _ZR, _ZR), :], acc.at[pl.ds(s * _ZR, _ZR), :])
    plsc.subcore_barrier()

    rows = [rows0, rows1]
    sems = [gsem0, gsem1]
    ssems = [ssem0, ssem1]
    sdesc = [None, None]
    for half in range(2):
        pltpu.sync_copy(srcp.at[w, pl.ds(half * hh, hh)], idx_s)
        pltpu.sync_copy(dstp.at[w, pl.ds(half * hh, hh)], idx_d)
        descs = [None, None]
        if sdesc[0] is not None:
            sdesc[0].wait()
            sdesc[0] = None
        descs[0] = pltpu.async_copy(table.at[idx_s.at[0]], rows[0], sems[0])
        for j in range(hh):
            b = j % 2
            nb = (j + 1) % 2
            if j + 1 < hh:
                if sdesc[nb] is not None:
                    sdesc[nb].wait()
                descs[nb] = pltpu.async_copy(table.at[idx_s.at[j + 1]], rows[nb], sems[nb])
            descs[b].wait()
            sdesc[b] = pltpu.async_copy(rows[b], acc.at[idx_d.at[j]], ssems[b], add=True)
        for b in range(2):
            if sdesc[b] is not None:
                sdesc[b].wait()
                sdesc[b] = None

    plsc.subcore_barrier()
    pltpu.sync_copy(acc.at[pl.ds(s * _ZR, _ZR), :],
                    out.at[pl.ds(c * _NPAD + s * _ZR, _ZR), :])


def _deg_body(dstp, zer1, out, idx_d, ones_v, acc1):
    c = lax.axis_index("c")
    s = lax.axis_index("s")
    w = s * _NC + c
    cpw = idx_d.shape[0]
    pltpu.sync_copy(zer1.at[pl.ds(s * _ZR, _ZR)], acc1.at[pl.ds(s * _ZR, _ZR)])
    pltpu.sync_copy(dstp.at[w], idx_d)
    for i in range(_C // 16):
        ones_v[pl.ds(i * 16, 16)] = jnp.ones((16,), jnp.float32)
    plsc.subcore_barrier()

    def step(j, carry):
        pltpu.sync_copy(ones_v, acc1.at[idx_d.at[j]], add=True)
        return carry

    lax.fori_loop(0, cpw, step, 0)
    plsc.subcore_barrier()

    @pl.when(s == 0)
    def _():
        pltpu.sync_copy(acc1, out.at[pl.ds(c * _NPAD, _NPAD)])


def _make_seg(cpw):
    return pl.kernel(
        _seg_body,
        out_type=jax.ShapeDtypeStruct((_NC * _NPAD, _WP), jnp.float32),
        mesh=_mesh,
        scratch_types=[
            pltpu.VMEM((cpw // 2, _C), jnp.int32),
            pltpu.VMEM((cpw // 2, _C), jnp.int32),
            pltpu.VMEM((_C, _WP), jnp.float32),
            pltpu.VMEM((_C, _WP), jnp.float32),
            pltpu.VMEM_SHARED((_NPAD, _WP), jnp.float32),
            pltpu.SemaphoreType.DMA,
            pltpu.SemaphoreType.DMA,
            pltpu.SemaphoreType.DMA,
            pltpu.SemaphoreType.DMA,
        ],
    )


def _make_deg(cpw):
    return pl.kernel(
        _deg_body,
        out_type=jax.ShapeDtypeStruct((_NC * _NPAD,), jnp.float32),
        mesh=_mesh,
        scratch_types=[
            pltpu.VMEM((cpw, _C), jnp.int32),
            pltpu.VMEM((_C,), jnp.float32),
            pltpu.VMEM_SHARED((_NPAD,), jnp.float32),
        ],
    )


_NB, _R = 10, _N // 10


def _k0_body(x_ref, w_ref, degp_ref, tab_ref, dinv_ref):
    deg = degp_ref[0] + degp_ref[1] + 1.0
    dinv = lax.rsqrt(deg)
    w = w_ref[...]
    h0 = jnp.dot(x_ref[0], w, preferred_element_type=jnp.float32, precision=lax.Precision.HIGHEST)
    h1 = jnp.dot(x_ref[1], w, preferred_element_type=jnp.float32, precision=lax.Precision.HIGHEST)
    t32 = jnp.concatenate([h0 * dinv, h1 * dinv], axis=1)
    tab_ref[...] = jnp.concatenate([t32, t32, t32, t32], axis=1)
    dinv_ref[...] = dinv


def _kmid_body(sp_ref, tp_ref, dinv_ref, b_ref, wbd_ref, out_ref):
    dinv = dinv_ref[...]
    s32 = (sp_ref[0] + sp_ref[1] + tp_ref[...])[:, :_W2]
    feat = dinv * s32 + b_ref[...]
    h = jnp.dot(feat, wbd_ref[...], preferred_element_type=jnp.float32, precision=lax.Precision.HIGHEST) * dinv
    out_ref[...] = jnp.concatenate([h, h, h, h], axis=1)


def _k4_body(sp_ref, tp_ref, dinv_ref, b_ref, wg1_ref, bg1_ref, wg2_ref, bg2_ref,
             wsrc_ref, wact_ref, wdst_ref, wg_ref, be1_ref, we2_ref, be2_ref,
             we3_ref, be3_ref, agent_ref, v_ref, feat_ref):
    s32 = (sp_ref[0, :_N, :] + sp_ref[1, :_N, :] + tp_ref[...])[:, :_W2]
    feat = dinv_ref[...] * s32 + b_ref[...]
    feat_ref[...] = feat
    m = jnp.sum(feat, axis=0, keepdims=True) * (1.0 / _N)
    g = jnp.concatenate([m[:, :_H], m[:, _H:]], axis=0)
    g1 = jnp.tanh(jnp.dot(g, wg1_ref[...], preferred_element_type=jnp.float32, precision=lax.Precision.HIGHEST) + bg1_ref[...])
    g2 = jnp.tanh(jnp.dot(g1, wg2_ref[...], preferred_element_type=jnp.float32, precision=lax.Precision.HIGHEST) + bg2_ref[...])
    gterm = jnp.dot(g2, wg_ref[...], preferred_element_type=jnp.float32, precision=lax.Precision.HIGHEST)

    srows, drows = [], []
    for b in range(_B):
        for k in range(8):
            si = agent_ref[b, k, 0]
            di = agent_ref[b, k, 1]
            srows.append(feat_ref[pl.ds(si, 1), pl.ds(b * _H, _H)])
            drows.append(feat_ref[pl.ds(di, 1), pl.ds(b * _H, _H)])
    src_f = jnp.concatenate(srows, axis=0)
    dst_f = jnp.concatenate(drows, axis=0)
    gt = jnp.concatenate(
        [jnp.broadcast_to(gterm[0:1, :], (8, _H)),
         jnp.broadcast_to(gterm[1:2, :], (8, _H))], axis=0)
    p = (jnp.dot(src_f, wsrc_ref[...], preferred_element_type=jnp.float32, precision=lax.Precision.HIGHEST)
         + jnp.dot(dst_f, wdst_ref[...], preferred_element_type=jnp.float32, precision=lax.Precision.HIGHEST)
         + gt + be1_ref[...])
    wact = wact_ref[...]
    vcols = []
    for t in range(3):
        h1 = jnp.tanh(p + wact[t:t + 1, :])
        h2 = jnp.tanh(jnp.dot(h1, we2_ref[...], preferred_element_type=jnp.float32, precision=lax.Precision.HIGHEST) + be2_ref[...])
        vcols.append(jnp.dot(h2, we3_ref[...], preferred_element_type=jnp.float32, precision=lax.Precision.HIGHEST) + be3_ref[...])
    v_ref[...] = jnp.concatenate(vcols, axis=1)


def kernel(x, edge_index, agent_edges, edge_actions, W_gcn, b_gcn, Wg1, bg1,
           Wg2, bg2, We1, be1, We2, be2, We3, be3):
    f32 = jnp.float32
    E = edge_index.shape[1]
    cpw = -(-E // (_NW * _C))
    cpw = -(-cpw // 8) * 8
    EP = _NW * cpw * _C
    padn = EP - E
    dum = _NPAD - _N

    pidx = jnp.arange(padn, dtype=jnp.int32)
    srcp = jnp.concatenate([edge_index[0], pidx % 16]).reshape(_NW, cpw, _C)
    dstp = jnp.concatenate([edge_index[1], _N + (pidx % dum)]).reshape(_NW, cpw, _C)

    zer1 = jnp.zeros((_NPAD,), f32)
    zer = jnp.zeros((_NPAD, _WP), f32)

    deg = _make_deg(cpw)(dstp, zer1).reshape(_NC, _NPAD)
    degp = deg[:, :_N, None]

    k0 = pl.pallas_call(
        _k0_body,
        grid=(_NB,),
        in_specs=[
            pl.BlockSpec((_B, _R, _D), lambda i: (0, i, 0)),
            pl.BlockSpec((_D, _H), lambda i: (0, 0)),
            pl.BlockSpec((_NC, _R, 1), lambda i: (0, i, 0)),
        ],
        out_specs=[
            pl.BlockSpec((_R, _WP), lambda i: (i, 0)),
            pl.BlockSpec((_R, 1), lambda i: (i, 0)),
        ],
        out_shape=[
            jax.ShapeDtypeStruct((_N, _WP), f32),
            jax.ShapeDtypeStruct((_N, 1), f32),
        ],
    )
    table, dinv = k0(x, W_gcn[0], degp)

    seg = _make_seg(cpw)
    kmid = pl.pallas_call(
        _kmid_body,
        grid=(_NB,),
        in_specs=[
            pl.BlockSpec((_NC, _R, _WP), lambda i: (0, i, 0)),
            pl.BlockSpec((_R, _WP), lambda i: (i, 0)),
            pl.BlockSpec((_R, 1), lambda i: (i, 0)),
            pl.BlockSpec((1, _W2), lambda i: (0, 0)),
            pl.BlockSpec((_W2, _W2), lambda i: (0, 0)),
        ],
        out_specs=pl.BlockSpec((_R, _WP), lambda i: (i, 0)),
        out_shape=jax.ShapeDtypeStruct((_N, _WP), f32),
    )

    def run_seg(tab):
        return seg(tab, srcp, dstp, zer).reshape(_NC, _NPAD, _WP)

    for l in range(1, 4):
        sp = run_seg(table)
        wbd = jnp.zeros((_W2, _W2), f32).at[:_H, :_H].set(W_gcn[l]).at[_H:, _H:].set(W_gcn[l])
        b2 = jnp.concatenate([b_gcn[l - 1], b_gcn[l - 1]])[None, :]
        table = kmid(sp, table, dinv, b2, wbd)

    spf = seg(table, srcp, dstp, zer).reshape(_NC, _NPAD, _WP)
    b2 = jnp.concatenate([b_gcn[3], b_gcn[3]])[None, :]

    k4 = pl.pallas_call(
        _k4_body,
        in_specs=[
            pl.BlockSpec((_NC, _NPAD, _WP), lambda: (0, 0, 0)),
            pl.BlockSpec((_N, _WP), lambda: (0, 0)),
            pl.BlockSpec((_N, 1), lambda: (0, 0)),
            pl.BlockSpec((1, _W2), lambda: (0, 0)),
            pl.BlockSpec((_H, 2 * _H), lambda: (0, 0)),
            pl.BlockSpec((1, 2 * _H), lambda: (0, 0)),
            pl.BlockSpec((2 * _H, _GDIM), lambda: (0, 0)),
            pl.BlockSpec((1, _GDIM), lambda: (0, 0)),
            pl.BlockSpec((_H, _H), lambda: (0, 0)),
            pl.BlockSpec((3, _H), lambda: (0, 0)),
            pl.BlockSpec((_H, _H), lambda: (0, 0)),
            pl.BlockSpec((_GDIM, _H), lambda: (0, 0)),
            pl.BlockSpec((1, _H), lambda: (0, 0)),
            pl.BlockSpec((_H, 8), lambda: (0, 0)),
            pl.BlockSpec((1, 8), lambda: (0, 0)),
            pl.BlockSpec((8, 1), lambda: (0, 0)),
            pl.BlockSpec((1, 1), lambda: (0, 0)),
            pl.BlockSpec(memory_space=pltpu.SMEM),
        ],
        out_specs=pl.BlockSpec((2 * 8, 3), lambda: (0, 0)),
        out_shape=jax.ShapeDtypeStruct((2 * 8, 3), f32),
        scratch_shapes=[pltpu.VMEM((_N, _W2), f32)],
    )
    v = k4(spf, table, dinv, b2, Wg1, bg1[None, :], Wg2, bg2[None, :],
           We1[:_H], We1[_H:_H + 3], We1[_H + 3:2 * _H + 3], We1[2 * _H + 3:],
           be1[None, :], We2, be2[None, :], We3, be3[None, :], agent_edges)

    v = v.reshape(_B, 8, 3)
    turn = jnp.arange(3, dtype=jnp.int32)[None, :]
    outs = []
    for bi in range(_B):
        disc = turn * 5 + edge_actions[bi][:, None]
        outs.append(jnp.zeros((15,), f32).at[disc.ravel()].set(v[bi].ravel()))
    return jnp.stack(outs, axis=0)
